# baseline (speedup 1.0000x reference)
import functools

import jax
import jax.numpy as jnp
from jax import lax
from jax.experimental import pallas as pl
from jax.experimental.pallas import tpu as pltpu
from jax.experimental.pallas import tpu_sc as plsc

B = 16384
K = 128
NC = 2
NS = 16
NW = NC * NS
BPW = B // NW
CH = 128
NCHUNK = BPW // CH
NSLOT = 2

_mesh = plsc.VectorSubcoreMesh(core_axis_name="c", subcore_axis_name="s")


@functools.partial(
    pl.kernel,
    out_type=jax.ShapeDtypeStruct((B,), jnp.float32),
    mesh=_mesh,
    compiler_params=pltpu.CompilerParams(needs_layout_passes=False,
                                        skip_device_barrier=True),
    scratch_types=[
        pltpu.VMEM((NCHUNK, CH), jnp.int32),
        pltpu.VMEM((NCHUNK, CH), jnp.int32),
        pltpu.VMEM((NSLOT, CH, K), jnp.float32),
        pltpu.VMEM((NSLOT, CH, K), jnp.float32),
        pltpu.VMEM((NCHUNK, CH), jnp.float32),
        pltpu.VMEM((NCHUNK, CH), jnp.float32),
        pltpu.VMEM((BPW,), jnp.float32),
        pltpu.VMEM((16,), jnp.float32),
        pltpu.SemaphoreType.DMA,
        pltpu.SemaphoreType.DMA,
        pltpu.SemaphoreType.DMA,
        pltpu.SemaphoreType.DMA,
    ],
)
def _bias_svd_sc(u_hbm, i_hbm, p_hbm, q_hbm, bu_hbm, bi_hbm, mu_hbm,
                 out_hbm, u_idx, i_idx, p_buf, q_buf, bu_buf, bi_buf,
                 out_buf, mu_v, sem0, sem1, sem2, semx):
    wid = lax.axis_index("s") * NC + lax.axis_index("c")
    base = wid * BPW
    sems = (sem0, sem1, sem2)

    start = [pltpu.async_copy(mu_hbm, mu_v.at[pl.ds(0, 1)], semx)]
    for c in range(NCHUNK):
        start.append(pltpu.async_copy(
            u_hbm.at[pl.ds(base + c * CH, CH)], u_idx.at[c], semx))
        start.append(pltpu.async_copy(
            i_hbm.at[pl.ds(base + c * CH, CH)], i_idx.at[c], semx))
    for h in start:
        h.wait()

    def issue(c, s):
        return (
            pltpu.async_copy(p_hbm.at[u_idx.at[c]], p_buf.at[s], sems[s]),
            pltpu.async_copy(q_hbm.at[i_idx.at[c]], q_buf.at[s], sems[s]),
        )

    handles = {0: issue(0, 0)}

    bias_handles = []
    for c in range(NCHUNK):
        bias_handles.append(
            pltpu.async_copy(bu_hbm.at[u_idx.at[c]], bu_buf.at[c], semx))
        bias_handles.append(
            pltpu.async_copy(bi_hbm.at[i_idx.at[c]], bi_buf.at[c], semx))

    lane = lax.iota(jnp.int32, 16)
    mu_s = mu_v[...][0]

    for c in range(NCHUNK):
        s = c % NSLOT
        if c + 1 < NCHUNK:
            handles[c + 1] = issue(c + 1, (c + 1) % NSLOT)
        if c == 0:
            for h in bias_handles:
                h.wait()
        for h in handles.pop(c):
            h.wait()

        pc, qc = p_buf.at[s], q_buf.at[s]
        buc, bic = bu_buf.at[c], bi_buf.at[c]

        def group_body(g, carry, c=c, pc=pc, qc=qc, buc=buc, bic=bic):
            r0 = g * 16
            o = buc[pl.ds(r0, 16)] + bic[pl.ds(r0, 16)] + mu_s
            for rr in range(16):
                acc = pc[r0 + rr, pl.ds(0, 16)] * qc[r0 + rr, pl.ds(0, 16)]
                for kc in range(1, K // 16):
                    acc = acc + (pc[r0 + rr, pl.ds(kc * 16, 16)]
                                 * qc[r0 + rr, pl.ds(kc * 16, 16)])
                o = o + acc
            out_buf[pl.ds(c * CH + r0, 16)] = o
            return carry

        lax.fori_loop(0, CH // 16, group_body, 0)

    pltpu.sync_copy(out_buf, out_hbm.at[pl.ds(base, BPW)])


def kernel(u, i, P, Q, bu, bi, mu):
    return _bias_svd_sc(u, i, P, Q, bu.reshape(-1), bi.reshape(-1), mu)


# speedup vs baseline: 1.0925x; 1.0925x over previous
import functools

import jax
import jax.numpy as jnp
from jax import lax
from jax.experimental import pallas as pl
from jax.experimental.pallas import tpu as pltpu
from jax.experimental.pallas import tpu_sc as plsc

B = 16384
K = 128
NC = 2
NS = 16
NW = NC * NS
BPW = B // NW
CH = 128
NCHUNK = BPW // CH
NSLOT = 2

_mesh = plsc.VectorSubcoreMesh(core_axis_name="c", subcore_axis_name="s")


@functools.partial(
    pl.kernel,
    out_type=jax.ShapeDtypeStruct((B,), jnp.float32),
    mesh=_mesh,
    compiler_params=pltpu.CompilerParams(needs_layout_passes=False,
                                        skip_device_barrier=True),
    scratch_types=[
        pltpu.VMEM((NCHUNK, CH), jnp.int32),
        pltpu.VMEM((NCHUNK, CH), jnp.int32),
        pltpu.VMEM((NSLOT, CH, K), jnp.float32),
        pltpu.VMEM((NSLOT, CH, K), jnp.float32),
        pltpu.VMEM((NCHUNK, CH), jnp.float32),
        pltpu.VMEM((NCHUNK, CH), jnp.float32),
        pltpu.VMEM((BPW,), jnp.float32),
        pltpu.VMEM((16,), jnp.float32),
        pltpu.SemaphoreType.DMA,
        pltpu.SemaphoreType.DMA,
        pltpu.SemaphoreType.DMA,
        pltpu.SemaphoreType.DMA,
    ],
)
def _bias_svd_sc(u_hbm, i_hbm, p_hbm, q_hbm, bu_hbm, bi_hbm, mu_hbm,
                 out_hbm, u_idx, i_idx, p_buf, q_buf, bu_buf, bi_buf,
                 out_buf, mu_v, sem0, sem1, sem2, semx):
    wid = lax.axis_index("s") * NC + lax.axis_index("c")
    base = wid * BPW
    sems = (sem0, sem1, sem2)

    start = [pltpu.async_copy(mu_hbm, mu_v.at[pl.ds(0, 1)], semx)]
    for c in range(NCHUNK):
        start.append(pltpu.async_copy(
            u_hbm.at[pl.ds(base + c * CH, CH)], u_idx.at[c], semx))
        start.append(pltpu.async_copy(
            i_hbm.at[pl.ds(base + c * CH, CH)], i_idx.at[c], semx))
    for h in start:
        h.wait()

    def issue(c, s):
        return (
            pltpu.async_copy(p_hbm.at[u_idx.at[c]], p_buf.at[s], sems[s]),
            pltpu.async_copy(q_hbm.at[i_idx.at[c]], q_buf.at[s], sems[s]),
        )

    handles = {0: issue(0, 0)}

    bias_handles = []
    for c in range(NCHUNK):
        bias_handles.append(
            pltpu.async_copy(bu_hbm.at[u_idx.at[c]], bu_buf.at[c], semx))
        bias_handles.append(
            pltpu.async_copy(bi_hbm.at[i_idx.at[c]], bi_buf.at[c], semx))

    lane = lax.iota(jnp.int32, 16)
    mu_s = mu_v[...][0]

    for c in range(NCHUNK):
        s = c % NSLOT
        if c + 1 < NCHUNK:
            handles[c + 1] = issue(c + 1, (c + 1) % NSLOT)
        if c == 0:
            for h in bias_handles:
                h.wait()
        for h in handles.pop(c):
            h.wait()

        pc, qc = p_buf.at[s], q_buf.at[s]
        buc, bic = bu_buf.at[c], bi_buf.at[c]

        def group_body(g, carry, c=c, pc=pc, qc=qc, buc=buc, bic=bic):
            r0 = g * 16
            o = buc[pl.ds(r0, 16)] + bic[pl.ds(r0, 16)] + mu_s
            for rr in range(16):
                prods = [pc[r0 + rr, pl.ds(kc * 16, 16)]
                         * qc[r0 + rr, pl.ds(kc * 16, 16)]
                         for kc in range(K // 16)]
                while len(prods) > 1:
                    prods = [prods[j] + prods[j + 1]
                             for j in range(0, len(prods), 2)]
                dot_r = jnp.sum(prods[0])
                o = jnp.where(lane == rr, o + dot_r, o)
            out_buf[pl.ds(c * CH + r0, 16)] = o
            return carry

        lax.fori_loop(0, CH // 16, group_body, 0)

    pltpu.sync_copy(out_buf, out_hbm.at[pl.ds(base, BPW)])


def kernel(u, i, P, Q, bu, bi, mu):
    return _bias_svd_sc(u, i, P, Q, bu.reshape(-1), bi.reshape(-1), mu)


# speedup vs baseline: 1.2386x; 1.1337x over previous
import functools

import jax
import jax.numpy as jnp
from jax import lax
from jax.experimental import pallas as pl
from jax.experimental.pallas import tpu as pltpu
from jax.experimental.pallas import tpu_sc as plsc

B = 16384
K = 128
NC = 2
NS = 16
NW = NC * NS
BPW = B // NW
CH = 128
NCHUNK = BPW // CH
NSLOT = 2

_mesh = plsc.VectorSubcoreMesh(core_axis_name="c", subcore_axis_name="s")


@functools.partial(
    pl.kernel,
    out_type=jax.ShapeDtypeStruct((B,), jnp.float32),
    mesh=_mesh,
    compiler_params=pltpu.CompilerParams(needs_layout_passes=False,
                                        skip_device_barrier=True),
    scratch_types=[
        pltpu.VMEM((NCHUNK, CH), jnp.int32),
        pltpu.VMEM((NCHUNK, CH), jnp.int32),
        pltpu.VMEM((NSLOT, CH, K), jnp.float32),
        pltpu.VMEM((NSLOT, CH, K), jnp.float32),
        pltpu.VMEM((NCHUNK, CH), jnp.float32),
        pltpu.VMEM((NCHUNK, CH), jnp.float32),
        pltpu.VMEM((BPW,), jnp.float32),
        pltpu.VMEM((16,), jnp.float32),
        pltpu.SemaphoreType.DMA,
        pltpu.SemaphoreType.DMA,
        pltpu.SemaphoreType.DMA,
        pltpu.SemaphoreType.DMA,
    ],
)
def _bias_svd_sc(u_hbm, i_hbm, p_hbm, q_hbm, bu_hbm, bi_hbm, mu_hbm,
                 out_hbm, u_idx, i_idx, p_buf, q_buf, bu_buf, bi_buf,
                 out_buf, mu_v, sem0, sem1, sem2, semx):
    wid = lax.axis_index("s") * NC + lax.axis_index("c")
    base = wid * BPW
    sems = (sem0, sem1, sem2)

    start = [pltpu.async_copy(mu_hbm, mu_v.at[pl.ds(0, 1)], semx)]
    for c in range(NCHUNK):
        start.append(pltpu.async_copy(
            u_hbm.at[pl.ds(base + c * CH, CH)], u_idx.at[c], semx))
        start.append(pltpu.async_copy(
            i_hbm.at[pl.ds(base + c * CH, CH)], i_idx.at[c], semx))
    for h in start:
        h.wait()

    def issue(c, s):
        return (
            pltpu.async_copy(p_hbm.at[u_idx.at[c]], p_buf.at[s], sems[s]),
            pltpu.async_copy(q_hbm.at[i_idx.at[c]], q_buf.at[s], sems[s]),
        )

    handles = {0: issue(0, 0)}

    bias_handles = []
    for c in range(NCHUNK):
        bias_handles.append(
            pltpu.async_copy(bu_hbm.at[u_idx.at[c]], bu_buf.at[c], semx))
        bias_handles.append(
            pltpu.async_copy(bi_hbm.at[i_idx.at[c]], bi_buf.at[c], semx))

    lane = lax.iota(jnp.int32, 16)
    mu_s = mu_v[...][0]

    for c in range(NCHUNK):
        s = c % NSLOT
        if c + 1 < NCHUNK:
            handles[c + 1] = issue(c + 1, (c + 1) % NSLOT)
        if c == 0:
            for h in bias_handles:
                h.wait()
        for h in handles.pop(c):
            h.wait()

        pc, qc = p_buf.at[s], q_buf.at[s]
        buc, bic = bu_buf.at[c], bi_buf.at[c]

        @plsc.parallel_loop(0, CH, step=16)
        def group_body(r0, c=c, pc=pc, qc=qc, buc=buc, bic=bic):
            o = buc[pl.ds(r0, 16)] + bic[pl.ds(r0, 16)] + mu_s
            for rr in range(16):
                acc = pc[r0 + rr, pl.ds(0, 16)] * qc[r0 + rr, pl.ds(0, 16)]
                for kc in range(1, K // 16):
                    acc = acc + (pc[r0 + rr, pl.ds(kc * 16, 16)]
                                 * qc[r0 + rr, pl.ds(kc * 16, 16)])
                dot_r = jnp.sum(acc)
                o = jnp.where(lane == rr, o + dot_r, o)
            out_buf[pl.ds(c * CH + r0, 16)] = o

    pltpu.sync_copy(out_buf, out_hbm.at[pl.ds(base, BPW)])


def kernel(u, i, P, Q, bu, bi, mu):
    return _bias_svd_sc(u, i, P, Q, bu.reshape(-1), bi.reshape(-1), mu)
